# baseline (speedup 1.0000x reference)
import functools

import jax
import jax.numpy as jnp
from jax import lax
from jax.experimental import pallas as pl
from jax.experimental.pallas import tpu as pltpu
from jax.experimental.pallas import tpu_sc as plsc

f32 = jnp.float32
i32 = jnp.int32

BOHR = 0.5291772105638411
HA = 27.211386024367243
S6, S8, A1, A2 = 1.0, 0.7875, 0.4289, 4.4407
KCN = 16.0
SQRT3 = 1.7320508075688772
EPS = 1.1920929e-07

MAX_Z, NREF = 95, 5
N_NODES, N_EDGES = 50000, 800000

NC, NS, L = 2, 16, 16
NW = NC * NS

EB = 1024
BLOCKS = 25
E_PER_W = EB * BLOCKS
E_PAD = NW * E_PER_W
N_PER_W = 1568
N_PAD = NW * N_PER_W
N_STRIPE = N_PAD // NS
WCOLS = 8
C6COLS = 16
C6ROWS = 9040

_INV_B2 = 1.0 / (BOHR * BOHR)

CN_O2, CN_C2 = 400.0, 625.0
E2_O2, E2_C2 = 3025.0, 3600.0


def _rsqrt(r):
    i = lax.bitcast_convert_type(r, i32)
    i = 0x5F3759DF - (i >> 1)
    y = lax.bitcast_convert_type(i, f32)
    for _ in range(3):
        y = y * (1.5 - 0.5 * r * y * y)
    return y


def _smooth_r2(r2, o2, c2):
    d = c2 - r2
    inner = d * d * (c2 + 2.0 * r2 - 3.0 * o2) * (1.0 / ((c2 - o2) * (c2 - o2) * (c2 - o2)))
    inner = jnp.where(r2 < c2, inner, 0.0)
    return jnp.where(r2 < o2, 1.0, inner)


def _mesh():
    return plsc.VectorSubcoreMesh(
        core_axis_name="c", subcore_axis_name="s", num_cores=NC, num_subcores=NS
    )


AB = 1280
ABLOCKS = E_PER_W // AB
AROWS = AB // 128


def _phase_a(xyz_h, i0_h, i1_h, num_h, rcov_h, r4_h):
    buf_types = [
        pltpu.VMEM((AROWS, 128), i32),
        pltpu.VMEM((AROWS, 128), i32),
        pltpu.VMEM((AB * 3,), f32),
        pltpu.VMEM((AROWS, 128), f32),
        pltpu.VMEM((AB,), f32),
        pltpu.VMEM((AB,), f32),
        pltpu.VMEM((AROWS, 128), i32),
    ]

    @functools.partial(
        pl.kernel,
        out_type=(
            jax.ShapeDtypeStruct((NC * N_PAD,), f32),
            jax.ShapeDtypeStruct((E_PAD,), f32),
            jax.ShapeDtypeStruct((E_PAD,), f32),
            jax.ShapeDtypeStruct((E_PAD // 128, 128), i32),
        ),
        mesh=_mesh(),
        compiler_params=pltpu.CompilerParams(needs_layout_passes=False, use_tc_tiling_on_sc=False),
        scratch_types=buf_types + buf_types + [
            pltpu.VMEM((N_PAD,), i32),
            pltpu.VMEM((96,), f32),
            pltpu.VMEM((96,), f32),
            pltpu.VMEM((96,), f32),
            pltpu.VMEM((N_STRIPE,), f32),
            pltpu.VMEM_SHARED((N_PAD,), f32),
            pltpu.SemaphoreType.DMA,
            pltpu.SemaphoreType.DMA,
            pltpu.SemaphoreType.DMA,
        ],
    )
    def body(xyz_r, i0_r, i1_r, num_r, rcov_r, r4_r,
             cn_o, sq_o, r2_o, p_o, *refs):
        bufs0 = refs[0:7]
        bufs1 = refs[7:14]
        num_v, rcov_v, r4_v, sr4_v, zero_v, cn_sh, semL, semS, semO = refs[14:23]
        cid = lax.axis_index("c")
        sid = lax.axis_index("s")
        wid = cid * NS + sid
        iota = lax.iota(i32, L)

        pltpu.sync_copy(num_r, num_v)
        pltpu.sync_copy(rcov_r, rcov_v)
        pltpu.sync_copy(r4_r, r4_v)

        def sqrt_tab(i, _):
            t = r4_v[pl.ds(i * L, L)]
            sr4_v[pl.ds(i * L, L)] = t * _rsqrt(jnp.maximum(t, 1e-12))
            return 0

        lax.fori_loop(0, 96 // L, sqrt_tab, 0)

        def zero_blk(i, _):
            zero_v[pl.ds(i * L, L)] = jnp.zeros((L,), f32)
            return 0

        lax.fori_loop(0, N_STRIPE // L, zero_blk, 0)
        pltpu.sync_copy(zero_v, cn_sh.at[pl.ds(sid * N_STRIPE, N_STRIPE)])
        plsc.subcore_barrier()

        def fire_lin(b, B):
            i0b, i1b, xyzb = B[:3]
            base = wid * E_PER_W + b * AB
            rowb = wid * (E_PER_W // 128) + b * AROWS
            pltpu.async_copy(i0_r.at[pl.ds(rowb, AROWS)], i0b, semL)
            pltpu.async_copy(i1_r.at[pl.ds(rowb, AROWS)], i1b, semL)
            pltpu.async_copy(xyz_r.at[pl.ds(base * 3, AB * 3)], xyzb, semL)

        def wait_lin(B):
            i0b, i1b, xyzb = B[:3]
            pltpu.make_async_copy(i0_r.at[pl.ds(0, AROWS)], i0b, semL).wait()
            pltpu.make_async_copy(i1_r.at[pl.ds(0, AROWS)], i1b, semL).wait()
            pltpu.make_async_copy(xyz_r.at[pl.ds(0, AB * 3)], xyzb, semL).wait()

        def fire_out(b, B):
            sqb, r2b, pb = B[4], B[5], B[6]
            base = wid * E_PER_W + b * AB
            rowb = wid * (E_PER_W // 128) + b * AROWS
            pltpu.async_copy(sqb, sq_o.at[pl.ds(base, AB)], semO)
            pltpu.async_copy(r2b, r2_o.at[pl.ds(base, AB)], semO)
            pltpu.async_copy(pb, p_o.at[pl.ds(rowb, AROWS)], semO)

        def wait_out(B):
            sqb, r2b, pb = B[4], B[5], B[6]
            pltpu.make_async_copy(sqb, sq_o.at[pl.ds(0, AB)], semO).wait()
            pltpu.make_async_copy(r2b, r2_o.at[pl.ds(0, AB)], semO).wait()
            pltpu.make_async_copy(pb, p_o.at[pl.ds(0, AROWS)], semO).wait()

        def compute(B):
            i0b, i1b, xyzb, cnvb, sqb, r2b, pb = B
            for j in range(AROWS):
                def grp(k, _):
                    off = j * 128 + k * L
                    i0 = i0b[j, pl.ds(k * L, L)]
                    i1 = i1b[j, pl.ds(k * L, L)]
                    zi = plsc.load_gather(num_v, [i0])
                    zj = plsc.load_gather(num_v, [i1])
                    idx3 = (off + iota) * 3
                    xv = plsc.load_gather(xyzb, [idx3])
                    yv = plsc.load_gather(xyzb, [idx3 + 1])
                    zv = plsc.load_gather(xyzb, [idx3 + 2])
                    r2 = (xv * xv + yv * yv + zv * zv) * _INV_B2
                    rc = plsc.load_gather(rcov_v, [zi]) + plsc.load_gather(rcov_v, [zj])
                    sq = SQRT3 * plsc.load_gather(sr4_v, [zi]) * plsc.load_gather(sr4_v, [zj])
                    inv_dr = _rsqrt(jnp.maximum(r2, 1e-12))
                    cnt = 1.0 / (1.0 + jnp.exp(-KCN * (rc * inv_dr - 1.0)))
                    cne = jnp.where(r2 > 0.0, _smooth_r2(r2, CN_O2, CN_C2) * cnt, 0.0)
                    cnvb[j, pl.ds(k * L, L)] = cne
                    sqb[pl.ds(off, L)] = sq
                    r2b[pl.ds(off, L)] = r2
                    pb[j, pl.ds(k * L, L)] = zj * MAX_Z + zi
                    return 0

                lax.fori_loop(0, 128 // L, grp, 0)
                pltpu.async_copy(cnvb.at[j], cn_sh.at[i0b.at[j]], semS, add=True)
            for j in range(AROWS):
                pltpu.make_async_copy(cnvb.at[j], cn_sh.at[i0b.at[j]], semS).wait()

        fire_lin(0, bufs0)
        fire_lin(1, bufs1)

        def step(i, _):
            e2 = jnp.minimum(2 * i + 2, ABLOCKS - 1)
            e3 = jnp.minimum(2 * i + 3, ABLOCKS - 1)
            wait_lin(bufs0)

            @pl.when(i != 0)
            def _():
                wait_out(bufs0)

            compute(bufs0)
            fire_out(2 * i, bufs0)
            fire_lin(e2, bufs0)
            wait_lin(bufs1)

            @pl.when(i != 0)
            def _():
                wait_out(bufs1)

            compute(bufs1)
            fire_out(2 * i + 1, bufs1)
            fire_lin(e3, bufs1)
            return 0

        lax.fori_loop(0, ABLOCKS // 2, step, 0)
        wait_out(bufs0)
        wait_out(bufs1)
        wait_lin(bufs0)
        wait_lin(bufs1)
        plsc.subcore_barrier()
        pltpu.sync_copy(cn_sh.at[pl.ds(sid * N_STRIPE, N_STRIPE)], zero_v)
        pltpu.sync_copy(
            zero_v, cn_o.at[pl.ds(cid * N_PAD + sid * N_STRIPE, N_STRIPE)]
        )

    return body(xyz_h, i0_h, i1_h, num_h, rcov_h, r4_h)


def _phase_b(num_h, refcn_h, cn_h):
    @functools.partial(
        pl.kernel,
        out_type=jax.ShapeDtypeStruct((N_PAD * WCOLS,), f32),
        mesh=_mesh(),
        compiler_params=pltpu.CompilerParams(needs_layout_passes=False, use_tc_tiling_on_sc=False),
        scratch_types=[
            pltpu.VMEM((480,), f32),
            pltpu.VMEM((N_PER_W,), i32),
            pltpu.VMEM((N_PER_W,), f32),
            pltpu.VMEM((N_PER_W,), f32),
            pltpu.VMEM((N_PER_W * WCOLS,), f32),
        ],
    )
    def body(num_r, refcn_r, cn_r, w_o, refcn_v, nums_v, cna_v, cnb_v, w_v):
        cid = lax.axis_index("c")
        sid = lax.axis_index("s")
        wid = cid * NS + sid
        nbase = wid * N_PER_W
        pltpu.sync_copy(refcn_r, refcn_v)
        pltpu.sync_copy(num_r.at[pl.ds(nbase, N_PER_W)], nums_v)
        pltpu.sync_copy(cn_r.at[pl.ds(nbase, N_PER_W)], cna_v)
        pltpu.sync_copy(cn_r.at[pl.ds(N_PAD + nbase, N_PER_W)], cnb_v)
        iota = lax.iota(i32, L)

        def grp(i, _):
            nm = nums_v[pl.ds(i * L, L)]
            cn = cna_v[pl.ds(i * L, L)] + cnb_v[pl.ds(i * L, L)]
            base5 = nm * NREF
            wl = []
            norm = jnp.zeros((L,), f32)
            for r in range(NREF):
                rc = plsc.load_gather(refcn_v, [base5 + r])
                d = rc - cn
                wv = jnp.where(rc >= 0.0, jnp.exp(-4.0 * d * d), 0.0)
                norm = norm + wv
                wl.append(wv)
            rn = 1.0 / (norm + EPS)
            li = (i * L + iota) * WCOLS
            for r in range(NREF):
                plsc.store_scatter(w_v, [li + r], wl[r] * rn)
            return 0

        lax.fori_loop(0, N_PER_W // L, grp, 0)
        pltpu.sync_copy(w_v, w_o.at[pl.ds(wid * N_PER_W * WCOLS, N_PER_W * WCOLS)])

    return body(num_h, refcn_h, cn_h)


CB = 512
CBLOCKS = E_PER_W // CB
CBLK0, CBLK1 = 50, 50
CROWS = CB // 128


def _phase_c(i0_h, i1_h, p_h, sq_h, r2_h, w_h, c6_h):
    buf_types = [
        pltpu.VMEM((CROWS, 128), i32),
        pltpu.VMEM((CROWS, 128), i32),
        pltpu.VMEM((CROWS, 128), i32),
        pltpu.VMEM((CB,), f32),
        pltpu.VMEM((CB,), f32),
        pltpu.VMEM((CB, WCOLS), f32),
        pltpu.VMEM((CB, WCOLS), f32),
        pltpu.VMEM((CB, C6COLS), i32),
    ]

    @functools.partial(
        pl.kernel,
        out_type=jax.ShapeDtypeStruct((NW, L), f32),
        mesh=_mesh(),
        compiler_params=pltpu.CompilerParams(needs_layout_passes=False, use_tc_tiling_on_sc=False),
        scratch_types=buf_types + buf_types + [
            pltpu.VMEM((L,), f32),
            pltpu.VMEM((N_PAD // NS, WCOLS), f32),
            pltpu.VMEM((C6ROWS // NS, C6COLS), i32),
            pltpu.VMEM_SHARED((N_PAD, WCOLS), f32),
            pltpu.VMEM_SHARED((C6ROWS, C6COLS), i32),
            pltpu.SemaphoreType.DMA,
            pltpu.SemaphoreType.DMA,
        ],
    )
    def body(i0_r, i1_r, p_r, sq_r, r2_r, w_r, c6_r, ps_o, *refs):
        bufs0 = refs[0:8]
        bufs1 = refs[8:16]
        acc_v, w_st, c6_st, w_sh, c6_sh, sem0, sem1 = refs[16:23]
        cid = lax.axis_index("c")
        sid = lax.axis_index("s")
        wid = cid * NS + sid
        iota = lax.iota(i32, L)

        wrows = N_PAD // NS
        crows = C6ROWS // NS
        pltpu.sync_copy(w_r.at[pl.ds(sid * wrows, wrows)], w_st)
        pltpu.sync_copy(w_st, w_sh.at[pl.ds(sid * wrows, wrows)])
        pltpu.sync_copy(c6_r.at[pl.ds(sid * crows, crows)], c6_st)
        pltpu.sync_copy(c6_st, c6_sh.at[pl.ds(sid * crows, crows)])
        plsc.subcore_barrier()

        nb = jnp.where(cid == 0, CBLK0, CBLK1)
        tile_base = (cid * NS * CBLK0 + sid * nb) * CB

        def fire_lin(b, B):
            i0b, i1b, pb, sqb, r2b = B[:5]
            base = tile_base + b * CB
            rowb = tile_base // 128 + b * CROWS
            pltpu.async_copy(i0_r.at[pl.ds(rowb, CROWS)], i0b, sem0)
            pltpu.async_copy(i1_r.at[pl.ds(rowb, CROWS)], i1b, sem0)
            pltpu.async_copy(p_r.at[pl.ds(rowb, CROWS)], pb, sem0)
            pltpu.async_copy(sq_r.at[pl.ds(base, CB)], sqb, sem0)
            pltpu.async_copy(r2_r.at[pl.ds(base, CB)], r2b, sem0)

        def wait_lin(B):
            i0b, i1b, pb, sqb, r2b = B[:5]
            pltpu.make_async_copy(i0_r.at[pl.ds(0, CROWS)], i0b, sem0).wait()
            pltpu.make_async_copy(i1_r.at[pl.ds(0, CROWS)], i1b, sem0).wait()
            pltpu.make_async_copy(p_r.at[pl.ds(0, CROWS)], pb, sem0).wait()
            pltpu.make_async_copy(sq_r.at[pl.ds(0, CB)], sqb, sem0).wait()
            pltpu.make_async_copy(r2_r.at[pl.ds(0, CB)], r2b, sem0).wait()

        def fire_gath(B):
            i0b, i1b, pb = B[:3]
            wib, wjb, c6b = B[5:8]
            for j in range(CROWS):
                pltpu.async_copy(w_sh.at[i0b.at[j]], wib.at[pl.ds(j * 128, 128)], sem1)
                pltpu.async_copy(w_sh.at[i1b.at[j]], wjb.at[pl.ds(j * 128, 128)], sem1)
                pltpu.async_copy(c6_sh.at[pb.at[j]], c6b.at[pl.ds(j * 128, 128)], sem1)

        def wait_gath(B):
            wib, wjb, c6b = B[5:8]
            for j in range(CROWS):
                pltpu.make_async_copy(
                    w_r.at[pl.ds(0, 128)], wib.at[pl.ds(j * 128, 128)], sem1).wait()
                pltpu.make_async_copy(
                    w_r.at[pl.ds(0, 128)], wjb.at[pl.ds(j * 128, 128)], sem1).wait()
                pltpu.make_async_copy(
                    c6_r.at[pl.ds(0, 128)], c6b.at[pl.ds(j * 128, 128)], sem1).wait()

        def compute(B, acc):
            sqr, r2r = B[3:5]
            wiv, wjv, c6v = B[5:8]

            def grp(i, a):
                lane = i * L + iota
                wib = [plsc.load_gather(wiv, [lane, jnp.full((L,), bb, i32)])
                       for bb in range(NREF)]
                vals = []
                for wd in range(13):
                    w = plsc.load_gather(c6v, [lane, jnp.full((L,), wd, i32)])
                    vals.append(lax.bitcast_convert_type(w << 16, f32))
                    vals.append(lax.bitcast_convert_type(
                        w & jnp.int32(-65536), f32))
                c6val = jnp.zeros((L,), f32)
                for aa in range(NREF):
                    s = jnp.zeros((L,), f32)
                    for bb in range(NREF):
                        s = s + wib[bb] * vals[aa * NREF + bb]
                    wja = plsc.load_gather(wjv, [lane, jnp.full((L,), aa, i32)])
                    c6val = c6val + wja * s
                sq = sqr[pl.ds(i * L, L)]
                r2 = r2r[pl.ds(i * L, L)]
                qq = sq * sq
                r0 = A1 * sq + A2
                r0_2 = r0 * r0
                r0_6 = r0_2 * r0_2 * r0_2
                r2_2 = r2 * r2
                r2_3 = r2_2 * r2
                t6 = 1.0 / (r2_3 + r0_6)
                t8 = 1.0 / (r2_2 * r2_2 + r0_6 * r0_2)
                e = S6 * c6val * t6 + S8 * c6val * qq * t8
                return a + _smooth_r2(r2, E2_O2, E2_C2) * e

            return lax.fori_loop(0, CB // L, grp, acc)

        fire_lin(0, bufs0)
        wait_lin(bufs0)
        fire_gath(bufs0)
        fire_lin(1, bufs1)

        def step(i, acc):
            e2 = jnp.minimum(2 * i + 2, nb - 1)
            e3 = jnp.minimum(2 * i + 3, nb - 1)
            wait_lin(bufs1)
            fire_gath(bufs1)
            wait_gath(bufs0)
            acc = compute(bufs0, acc)
            fire_lin(e2, bufs0)
            wait_lin(bufs0)
            fire_gath(bufs0)
            wait_gath(bufs1)
            acc = compute(bufs1, acc)
            fire_lin(e3, bufs1)
            return acc

        acc = lax.fori_loop(0, nb // 2, step, jnp.zeros((L,), f32))
        wait_gath(bufs0)
        wait_lin(bufs1)
        acc_v[pl.ds(0, L)] = acc * (-0.5 * HA)
        pltpu.sync_copy(acc_v, ps_o.at[wid])

    return body(i0_h, i1_h, p_h, sq_h, r2_h, w_h, c6_h)


def kernel(dr_vec, r4r2, rcov, ref_cn_tab, ref_c6_tab, numbers, idx):
    dr_vec = dr_vec.astype(f32)
    pad_e = E_PAD - N_EDGES
    xyz_h = jnp.concatenate(
        [dr_vec.reshape(-1),
         jnp.broadcast_to(jnp.array([[1000.0, 0.0, 0.0]], f32),
                          (pad_e, 3)).reshape(-1)])
    idxp = jnp.concatenate(
        [idx.astype(i32), jnp.zeros((2, pad_e), i32)], axis=1)
    i0_h = idxp[0].reshape(E_PAD // 128, 128)
    i1_h = idxp[1].reshape(E_PAD // 128, 128)
    num_h = jnp.concatenate(
        [numbers.astype(i32), jnp.zeros((N_PAD - N_NODES,), i32)])
    rcov_h = jnp.concatenate([rcov.astype(f32), jnp.zeros((96 - MAX_Z,), f32)])
    r4_h = jnp.concatenate([r4r2.astype(f32), jnp.zeros((96 - MAX_Z,), f32)])
    refcn_h = jnp.concatenate(
        [ref_cn_tab.astype(f32).reshape(-1), jnp.full((480 - MAX_Z * NREF,), -1.0, f32)])
    c6_bf = jnp.concatenate(
        [ref_c6_tab.astype(jnp.bfloat16).reshape(MAX_Z * MAX_Z, NREF * NREF),
         jnp.zeros((MAX_Z * MAX_Z, 2 * C6COLS - NREF * NREF), jnp.bfloat16)],
        axis=1)
    c6_h = lax.bitcast_convert_type(
        c6_bf.reshape(MAX_Z * MAX_Z, C6COLS, 2), i32)
    c6_h = jnp.concatenate(
        [c6_h, jnp.zeros((C6ROWS - MAX_Z * MAX_Z, C6COLS), i32)], axis=0)

    cn_h, sq_h, r2_h, p_h = _phase_a(xyz_h, i0_h, i1_h, num_h, rcov_h, r4_h)
    w_flat = _phase_b(num_h, refcn_h, cn_h)
    w_h = w_flat.reshape(N_PAD, WCOLS)
    psum = _phase_c(i0_h, i1_h, p_h, sq_h, r2_h, w_h, c6_h)
    return jnp.sum(psum)


# speedup vs baseline: 10.5276x; 10.5276x over previous
import functools

import jax
import jax.numpy as jnp
from jax import lax
from jax.experimental import pallas as pl
from jax.experimental.pallas import tpu as pltpu
from jax.experimental.pallas import tpu_sc as plsc

f32 = jnp.float32
i32 = jnp.int32

BOHR = 0.5291772105638411
HA = 27.211386024367243
S6, S8, A1, A2 = 1.0, 0.7875, 0.4289, 4.4407
KCN = 16.0
SQRT3 = 1.7320508075688772
EPS = 1.1920929e-07

MAX_Z, NREF = 95, 5
N_NODES, N_EDGES = 50000, 800000

NC, NS, L = 2, 16, 16
NW = NC * NS

EB = 1024
BLOCKS = 25
E_PER_W = EB * BLOCKS
E_PAD = NW * E_PER_W
N_PER_W = 1568
N_PAD = NW * N_PER_W
N_STRIPE = N_PAD // NS
WCOLS = 8
C6COLS = 16
C6ROWS = 9040

_INV_B2 = 1.0 / (BOHR * BOHR)

CN_O2, CN_C2 = 400.0, 625.0
E2_O2, E2_C2 = 3025.0, 3600.0


def _rsqrt(r):
    i = lax.bitcast_convert_type(r, i32)
    i = 0x5F3759DF - (i >> 1)
    y = lax.bitcast_convert_type(i, f32)
    for _ in range(3):
        y = y * (1.5 - 0.5 * r * y * y)
    return y


def _smooth_r2(r2, o2, c2):
    d = c2 - r2
    inner = d * d * (c2 + 2.0 * r2 - 3.0 * o2) * (1.0 / ((c2 - o2) * (c2 - o2) * (c2 - o2)))
    inner = jnp.where(r2 < c2, inner, 0.0)
    return jnp.where(r2 < o2, 1.0, inner)


def _mesh():
    return plsc.VectorSubcoreMesh(
        core_axis_name="c", subcore_axis_name="s", num_cores=NC, num_subcores=NS
    )


AB = 1280
ABLOCKS = E_PER_W // AB
AROWS = AB // 128


def _phase_a(x_h, y_h, z_h, i0_h, i1_h, num_h, rcov_h, r4_h):
    buf_types = [
        pltpu.VMEM((AROWS, 128), i32),
        pltpu.VMEM((AROWS, 128), i32),
        pltpu.VMEM((AB,), f32),
        pltpu.VMEM((AB,), f32),
        pltpu.VMEM((AB,), f32),
        pltpu.VMEM((AROWS, 128), f32),
        pltpu.VMEM((AB,), f32),
        pltpu.VMEM((AB,), f32),
        pltpu.VMEM((AROWS, 128), i32),
    ]

    @functools.partial(
        pl.kernel,
        out_type=(
            jax.ShapeDtypeStruct((NC * N_PAD,), f32),
            jax.ShapeDtypeStruct((E_PAD,), f32),
            jax.ShapeDtypeStruct((E_PAD,), f32),
            jax.ShapeDtypeStruct((E_PAD // 128, 128), i32),
        ),
        mesh=_mesh(),
        compiler_params=pltpu.CompilerParams(needs_layout_passes=False, use_tc_tiling_on_sc=False),
        scratch_types=buf_types + buf_types + [
            pltpu.VMEM((N_PAD,), i32),
            pltpu.VMEM((96,), f32),
            pltpu.VMEM((96,), f32),
            pltpu.VMEM((96,), f32),
            pltpu.VMEM((N_STRIPE,), f32),
            pltpu.VMEM_SHARED((N_PAD,), f32),
            pltpu.SemaphoreType.DMA,
            pltpu.SemaphoreType.DMA,
            pltpu.SemaphoreType.DMA,
        ],
    )
    def body(x_r, y_r, z_r, i0_r, i1_r, num_r, rcov_r, r4_r,
             cn_o, sq_o, r2_o, p_o, *refs):
        bufs0 = refs[0:9]
        bufs1 = refs[9:18]
        num_v, rcov_v, r4_v, sr4_v, zero_v, cn_sh, semL, semS, semO = refs[18:27]
        cid = lax.axis_index("c")
        sid = lax.axis_index("s")
        wid = cid * NS + sid
        iota = lax.iota(i32, L)

        pltpu.sync_copy(num_r, num_v)
        pltpu.sync_copy(rcov_r, rcov_v)
        pltpu.sync_copy(r4_r, r4_v)

        def sqrt_tab(i, _):
            t = r4_v[pl.ds(i * L, L)]
            sr4_v[pl.ds(i * L, L)] = t * _rsqrt(jnp.maximum(t, 1e-12))
            return 0

        lax.fori_loop(0, 96 // L, sqrt_tab, 0)

        def zero_blk(i, _):
            zero_v[pl.ds(i * L, L)] = jnp.zeros((L,), f32)
            return 0

        lax.fori_loop(0, N_STRIPE // L, zero_blk, 0)
        pltpu.sync_copy(zero_v, cn_sh.at[pl.ds(sid * N_STRIPE, N_STRIPE)])
        plsc.subcore_barrier()

        def fire_lin(b, B):
            i0b, i1b, xb, yb, zb = B[:5]
            base = wid * E_PER_W + b * AB
            rowb = wid * (E_PER_W // 128) + b * AROWS
            pltpu.async_copy(i0_r.at[pl.ds(rowb, AROWS)], i0b, semL)
            pltpu.async_copy(i1_r.at[pl.ds(rowb, AROWS)], i1b, semL)
            pltpu.async_copy(x_r.at[pl.ds(base, AB)], xb, semL)
            pltpu.async_copy(y_r.at[pl.ds(base, AB)], yb, semL)
            pltpu.async_copy(z_r.at[pl.ds(base, AB)], zb, semL)

        def wait_lin(B):
            i0b, i1b, xb, yb, zb = B[:5]
            pltpu.make_async_copy(i0_r.at[pl.ds(0, AROWS)], i0b, semL).wait()
            pltpu.make_async_copy(i1_r.at[pl.ds(0, AROWS)], i1b, semL).wait()
            pltpu.make_async_copy(x_r.at[pl.ds(0, AB)], xb, semL).wait()
            pltpu.make_async_copy(y_r.at[pl.ds(0, AB)], yb, semL).wait()
            pltpu.make_async_copy(z_r.at[pl.ds(0, AB)], zb, semL).wait()

        def fire_out(b, B):
            sqb, r2b, pb = B[6], B[7], B[8]
            base = wid * E_PER_W + b * AB
            rowb = wid * (E_PER_W // 128) + b * AROWS
            pltpu.async_copy(sqb, sq_o.at[pl.ds(base, AB)], semO)
            pltpu.async_copy(r2b, r2_o.at[pl.ds(base, AB)], semO)
            pltpu.async_copy(pb, p_o.at[pl.ds(rowb, AROWS)], semO)

        def wait_out(B):
            sqb, r2b, pb = B[6], B[7], B[8]
            pltpu.make_async_copy(sqb, sq_o.at[pl.ds(0, AB)], semO).wait()
            pltpu.make_async_copy(r2b, r2_o.at[pl.ds(0, AB)], semO).wait()
            pltpu.make_async_copy(pb, p_o.at[pl.ds(0, AROWS)], semO).wait()

        def compute(B):
            i0b, i1b, xb, yb, zb, cnvb, sqb, r2b, pb = B
            for j in range(AROWS):
                def grp(k, _):
                    off = j * 128 + k * L
                    i0 = i0b[j, pl.ds(k * L, L)]
                    i1 = i1b[j, pl.ds(k * L, L)]
                    zi = plsc.load_gather(num_v, [i0])
                    zj = plsc.load_gather(num_v, [i1])
                    xv = xb[pl.ds(off, L)]
                    yv = yb[pl.ds(off, L)]
                    zv = zb[pl.ds(off, L)]
                    r2 = (xv * xv + yv * yv + zv * zv) * _INV_B2
                    rc = plsc.load_gather(rcov_v, [zi]) + plsc.load_gather(rcov_v, [zj])
                    sq = SQRT3 * plsc.load_gather(sr4_v, [zi]) * plsc.load_gather(sr4_v, [zj])
                    inv_dr = _rsqrt(jnp.maximum(r2, 1e-12))
                    cnt = 1.0 / (1.0 + jnp.exp(-KCN * (rc * inv_dr - 1.0)))
                    cne = jnp.where(r2 > 0.0, _smooth_r2(r2, CN_O2, CN_C2) * cnt, 0.0)
                    cnvb[j, pl.ds(k * L, L)] = cne
                    sqb[pl.ds(off, L)] = sq
                    r2b[pl.ds(off, L)] = r2
                    pb[j, pl.ds(k * L, L)] = zj * MAX_Z + zi
                    return 0

                lax.fori_loop(0, 128 // L, grp, 0)
                pltpu.async_copy(cnvb.at[j], cn_sh.at[i0b.at[j]], semS, add=True)
            for j in range(AROWS):
                pltpu.make_async_copy(cnvb.at[j], cn_sh.at[i0b.at[j]], semS).wait()

        fire_lin(0, bufs0)
        fire_lin(1, bufs1)

        def step(i, _):
            e2 = jnp.minimum(2 * i + 2, ABLOCKS - 1)
            e3 = jnp.minimum(2 * i + 3, ABLOCKS - 1)
            wait_lin(bufs0)

            @pl.when(i != 0)
            def _():
                wait_out(bufs0)

            compute(bufs0)
            fire_out(2 * i, bufs0)
            fire_lin(e2, bufs0)
            wait_lin(bufs1)

            @pl.when(i != 0)
            def _():
                wait_out(bufs1)

            compute(bufs1)
            fire_out(2 * i + 1, bufs1)
            fire_lin(e3, bufs1)
            return 0

        lax.fori_loop(0, ABLOCKS // 2, step, 0)
        wait_out(bufs0)
        wait_out(bufs1)
        wait_lin(bufs0)
        wait_lin(bufs1)
        plsc.subcore_barrier()
        pltpu.sync_copy(cn_sh.at[pl.ds(sid * N_STRIPE, N_STRIPE)], zero_v)
        pltpu.sync_copy(
            zero_v, cn_o.at[pl.ds(cid * N_PAD + sid * N_STRIPE, N_STRIPE)]
        )

    return body(x_h, y_h, z_h, i0_h, i1_h, num_h, rcov_h, r4_h)


def _phase_b(num_h, refcn_h, cn_h):
    @functools.partial(
        pl.kernel,
        out_type=jax.ShapeDtypeStruct((N_PAD * WCOLS,), f32),
        mesh=_mesh(),
        compiler_params=pltpu.CompilerParams(needs_layout_passes=False, use_tc_tiling_on_sc=False),
        scratch_types=[
            pltpu.VMEM((480,), f32),
            pltpu.VMEM((N_PER_W,), i32),
            pltpu.VMEM((N_PER_W,), f32),
            pltpu.VMEM((N_PER_W,), f32),
            pltpu.VMEM((N_PER_W * WCOLS,), f32),
        ],
    )
    def body(num_r, refcn_r, cn_r, w_o, refcn_v, nums_v, cna_v, cnb_v, w_v):
        cid = lax.axis_index("c")
        sid = lax.axis_index("s")
        wid = cid * NS + sid
        nbase = wid * N_PER_W
        pltpu.sync_copy(refcn_r, refcn_v)
        pltpu.sync_copy(num_r.at[pl.ds(nbase, N_PER_W)], nums_v)
        pltpu.sync_copy(cn_r.at[pl.ds(nbase, N_PER_W)], cna_v)
        pltpu.sync_copy(cn_r.at[pl.ds(N_PAD + nbase, N_PER_W)], cnb_v)
        iota = lax.iota(i32, L)

        def grp(i, _):
            nm = nums_v[pl.ds(i * L, L)]
            cn = cna_v[pl.ds(i * L, L)] + cnb_v[pl.ds(i * L, L)]
            base5 = nm * NREF
            wl = []
            norm = jnp.zeros((L,), f32)
            for r in range(NREF):
                rc = plsc.load_gather(refcn_v, [base5 + r])
                d = rc - cn
                wv = jnp.where(rc >= 0.0, jnp.exp(-4.0 * d * d), 0.0)
                norm = norm + wv
                wl.append(wv)
            rn = 1.0 / (norm + EPS)
            li = (i * L + iota) * WCOLS
            for r in range(NREF):
                plsc.store_scatter(w_v, [li + r], wl[r] * rn)
            return 0

        lax.fori_loop(0, N_PER_W // L, grp, 0)
        pltpu.sync_copy(w_v, w_o.at[pl.ds(wid * N_PER_W * WCOLS, N_PER_W * WCOLS)])

    return body(num_h, refcn_h, cn_h)


CB = 512
CBLOCKS = E_PER_W // CB
CBLK0, CBLK1 = 50, 50
CROWS = CB // 128


def _phase_c(i0_h, i1_h, p_h, sq_h, r2_h, w_h, c6_h):
    buf_types = [
        pltpu.VMEM((CROWS, 128), i32),
        pltpu.VMEM((CROWS, 128), i32),
        pltpu.VMEM((CROWS, 128), i32),
        pltpu.VMEM((CB,), f32),
        pltpu.VMEM((CB,), f32),
        pltpu.VMEM((CB, WCOLS), f32),
        pltpu.VMEM((CB, WCOLS), f32),
        pltpu.VMEM((CB, C6COLS), i32),
    ]

    @functools.partial(
        pl.kernel,
        out_type=jax.ShapeDtypeStruct((NW, L), f32),
        mesh=_mesh(),
        compiler_params=pltpu.CompilerParams(needs_layout_passes=False, use_tc_tiling_on_sc=False),
        scratch_types=buf_types + buf_types + [
            pltpu.VMEM((L,), f32),
            pltpu.VMEM((N_PAD // NS, WCOLS), f32),
            pltpu.VMEM((C6ROWS // NS, C6COLS), i32),
            pltpu.VMEM_SHARED((N_PAD, WCOLS), f32),
            pltpu.VMEM_SHARED((C6ROWS, C6COLS), i32),
            pltpu.SemaphoreType.DMA,
            pltpu.SemaphoreType.DMA,
        ],
    )
    def body(i0_r, i1_r, p_r, sq_r, r2_r, w_r, c6_r, ps_o, *refs):
        bufs0 = refs[0:8]
        bufs1 = refs[8:16]
        acc_v, w_st, c6_st, w_sh, c6_sh, sem0, sem1 = refs[16:23]
        cid = lax.axis_index("c")
        sid = lax.axis_index("s")
        wid = cid * NS + sid
        iota = lax.iota(i32, L)

        wrows = N_PAD // NS
        crows = C6ROWS // NS
        pltpu.sync_copy(w_r.at[pl.ds(sid * wrows, wrows)], w_st)
        pltpu.sync_copy(w_st, w_sh.at[pl.ds(sid * wrows, wrows)])
        pltpu.sync_copy(c6_r.at[pl.ds(sid * crows, crows)], c6_st)
        pltpu.sync_copy(c6_st, c6_sh.at[pl.ds(sid * crows, crows)])
        plsc.subcore_barrier()

        nb = jnp.where(cid == 0, CBLK0, CBLK1)
        tile_base = (cid * NS * CBLK0 + sid * nb) * CB

        def fire_lin(b, B):
            i0b, i1b, pb, sqb, r2b = B[:5]
            base = tile_base + b * CB
            rowb = tile_base // 128 + b * CROWS
            pltpu.async_copy(i0_r.at[pl.ds(rowb, CROWS)], i0b, sem0)
            pltpu.async_copy(i1_r.at[pl.ds(rowb, CROWS)], i1b, sem0)
            pltpu.async_copy(p_r.at[pl.ds(rowb, CROWS)], pb, sem0)
            pltpu.async_copy(sq_r.at[pl.ds(base, CB)], sqb, sem0)
            pltpu.async_copy(r2_r.at[pl.ds(base, CB)], r2b, sem0)

        def wait_lin(B):
            i0b, i1b, pb, sqb, r2b = B[:5]
            pltpu.make_async_copy(i0_r.at[pl.ds(0, CROWS)], i0b, sem0).wait()
            pltpu.make_async_copy(i1_r.at[pl.ds(0, CROWS)], i1b, sem0).wait()
            pltpu.make_async_copy(p_r.at[pl.ds(0, CROWS)], pb, sem0).wait()
            pltpu.make_async_copy(sq_r.at[pl.ds(0, CB)], sqb, sem0).wait()
            pltpu.make_async_copy(r2_r.at[pl.ds(0, CB)], r2b, sem0).wait()

        def fire_gath(B):
            i0b, i1b, pb = B[:3]
            wib, wjb, c6b = B[5:8]
            for j in range(CROWS):
                pltpu.async_copy(w_sh.at[i0b.at[j]], wib.at[pl.ds(j * 128, 128)], sem1)
                pltpu.async_copy(w_sh.at[i1b.at[j]], wjb.at[pl.ds(j * 128, 128)], sem1)
                pltpu.async_copy(c6_sh.at[pb.at[j]], c6b.at[pl.ds(j * 128, 128)], sem1)

        def wait_gath(B):
            wib, wjb, c6b = B[5:8]
            for j in range(CROWS):
                pltpu.make_async_copy(
                    w_r.at[pl.ds(0, 128)], wib.at[pl.ds(j * 128, 128)], sem1).wait()
                pltpu.make_async_copy(
                    w_r.at[pl.ds(0, 128)], wjb.at[pl.ds(j * 128, 128)], sem1).wait()
                pltpu.make_async_copy(
                    c6_r.at[pl.ds(0, 128)], c6b.at[pl.ds(j * 128, 128)], sem1).wait()

        def compute(B, acc):
            sqr, r2r = B[3:5]
            wiv, wjv, c6v = B[5:8]

            def grp(i, a):
                lane = i * L + iota
                wib = [plsc.load_gather(wiv, [lane, jnp.full((L,), bb, i32)])
                       for bb in range(NREF)]
                vals = []
                for wd in range(13):
                    w = plsc.load_gather(c6v, [lane, jnp.full((L,), wd, i32)])
                    vals.append(lax.bitcast_convert_type(w << 16, f32))
                    vals.append(lax.bitcast_convert_type(
                        w & jnp.int32(-65536), f32))
                c6val = jnp.zeros((L,), f32)
                for aa in range(NREF):
                    s = jnp.zeros((L,), f32)
                    for bb in range(NREF):
                        s = s + wib[bb] * vals[aa * NREF + bb]
                    wja = plsc.load_gather(wjv, [lane, jnp.full((L,), aa, i32)])
                    c6val = c6val + wja * s
                sq = sqr[pl.ds(i * L, L)]
                r2 = r2r[pl.ds(i * L, L)]
                qq = sq * sq
                r0 = A1 * sq + A2
                r0_2 = r0 * r0
                r0_6 = r0_2 * r0_2 * r0_2
                r2_2 = r2 * r2
                r2_3 = r2_2 * r2
                t6 = 1.0 / (r2_3 + r0_6)
                t8 = 1.0 / (r2_2 * r2_2 + r0_6 * r0_2)
                e = S6 * c6val * t6 + S8 * c6val * qq * t8
                return a + _smooth_r2(r2, E2_O2, E2_C2) * e

            return lax.fori_loop(0, CB // L, grp, acc)

        fire_lin(0, bufs0)
        wait_lin(bufs0)
        fire_gath(bufs0)
        fire_lin(1, bufs1)

        def step(i, acc):
            e2 = jnp.minimum(2 * i + 2, nb - 1)
            e3 = jnp.minimum(2 * i + 3, nb - 1)
            wait_lin(bufs1)
            fire_gath(bufs1)
            wait_gath(bufs0)
            acc = compute(bufs0, acc)
            fire_lin(e2, bufs0)
            wait_lin(bufs0)
            fire_gath(bufs0)
            wait_gath(bufs1)
            acc = compute(bufs1, acc)
            fire_lin(e3, bufs1)
            return acc

        acc = lax.fori_loop(0, nb // 2, step, jnp.zeros((L,), f32))
        wait_gath(bufs0)
        wait_lin(bufs1)
        acc_v[pl.ds(0, L)] = acc * (-0.5 * HA)
        pltpu.sync_copy(acc_v, ps_o.at[wid])

    return body(i0_h, i1_h, p_h, sq_h, r2_h, w_h, c6_h)


def kernel(dr_vec, r4r2, rcov, ref_cn_tab, ref_c6_tab, numbers, idx):
    dr_vec = dr_vec.astype(f32)
    pad_e = E_PAD - N_EDGES
    drp = jnp.concatenate(
        [dr_vec, jnp.broadcast_to(jnp.array([[1000.0, 0.0, 0.0]], f32),
                                  (pad_e, 3))], 0)
    x_h = drp[:, 0]
    y_h = drp[:, 1]
    z_h = drp[:, 2]
    idxp = jnp.concatenate(
        [idx.astype(i32), jnp.zeros((2, pad_e), i32)], axis=1)
    i0_h = idxp[0].reshape(E_PAD // 128, 128)
    i1_h = idxp[1].reshape(E_PAD // 128, 128)
    num_h = jnp.concatenate(
        [numbers.astype(i32), jnp.zeros((N_PAD - N_NODES,), i32)])
    rcov_h = jnp.concatenate([rcov.astype(f32), jnp.zeros((96 - MAX_Z,), f32)])
    r4_h = jnp.concatenate([r4r2.astype(f32), jnp.zeros((96 - MAX_Z,), f32)])
    refcn_h = jnp.concatenate(
        [ref_cn_tab.astype(f32).reshape(-1), jnp.full((480 - MAX_Z * NREF,), -1.0, f32)])
    c6_bf = jnp.concatenate(
        [ref_c6_tab.astype(jnp.bfloat16).reshape(MAX_Z * MAX_Z, NREF * NREF),
         jnp.zeros((MAX_Z * MAX_Z, 2 * C6COLS - NREF * NREF), jnp.bfloat16)],
        axis=1)
    c6_h = lax.bitcast_convert_type(
        c6_bf.reshape(MAX_Z * MAX_Z, C6COLS, 2), i32)
    c6_h = jnp.concatenate(
        [c6_h, jnp.zeros((C6ROWS - MAX_Z * MAX_Z, C6COLS), i32)], axis=0)

    cn_h, sq_h, r2_h, p_h = _phase_a(x_h, y_h, z_h, i0_h, i1_h, num_h, rcov_h, r4_h)
    w_flat = _phase_b(num_h, refcn_h, cn_h)
    w_h = w_flat.reshape(N_PAD, WCOLS)
    psum = _phase_c(i0_h, i1_h, p_h, sq_h, r2_h, w_h, c6_h)
    return jnp.sum(psum)
